# Initial kernel scaffold; baseline (speedup 1.0000x reference)
import jax
import jax.numpy as jnp
from jax.experimental import pallas as pl


def kernel(inputs, table):
    raise NotImplementedError("write your pallas kernel here")


# speedup vs baseline: 5.1514x; 5.1514x over previous
import functools

import jax
import jax.numpy as jnp
from jax import lax
from jax.experimental import pallas as pl
from jax.experimental.pallas import tpu as pltpu
from jax.experimental.pallas import tpu_sc as plsc

VOCAB = 100000
DIM = 128
BATCH = 4096
SEQ = 200

NC = 2
NS = 16
NW = NC * NS

B = BATCH * SEQ
B_PER_W = B // NW
CHUNK = 128
NCHUNK = B_PER_W // CHUNK

_mesh = plsc.VectorSubcoreMesh(core_axis_name="c", subcore_axis_name="s")


@functools.partial(
    pl.kernel,
    mesh=_mesh,
    out_type=jax.ShapeDtypeStruct((B, DIM), jnp.float32),
    scratch_types=[
        pltpu.VMEM((CHUNK,), jnp.int32),
        pltpu.VMEM((CHUNK, DIM), jnp.float32),
        pltpu.VMEM((16, DIM), jnp.float32),
        pltpu.SemaphoreType.DMA,
    ],
)
def _embed(idx_hbm, table_hbm, out_hbm, idx_v, rows_v, zeros_v, sem):
    wid = lax.axis_index("s") * NC + lax.axis_index("c")
    base = wid * B_PER_W

    zvec = jnp.zeros((16,), jnp.float32)

    def zinit(r, _):
        for c in range(DIM // 16):
            zeros_v[r, pl.ds(c * 16, 16)] = zvec
        return 0

    lax.fori_loop(0, 16, zinit, 0)

    def chunk_body(g, _):
        off = base + g * CHUNK
        pltpu.sync_copy(idx_hbm.at[pl.ds(off, CHUNK)], idx_v)
        pltpu.async_copy(table_hbm.at[idx_v], rows_v, sem).wait()
        pltpu.sync_copy(rows_v, out_hbm.at[pl.ds(off, CHUNK)])

        def min_groups(i, acc):
            iv = idx_v[pl.ds(i * 16, 16)]
            return jnp.minimum(acc, iv)

        idx_min = lax.fori_loop(
            0, CHUNK // 16, min_groups, jnp.full((16,), VOCAB, jnp.int32)
        )
        chunk_min = idx_min[0]
        for j in range(1, 16):
            chunk_min = jnp.minimum(chunk_min, idx_min[j])

        @pl.when(chunk_min == 0)
        def _():
            def fix_group(i, _):
                iv = idx_v[pl.ds(i * 16, 16)]
                is_pad = iv == 0
                group_min = iv[0]
                first = jnp.where(iv[15] == 0, 15, 16)
                for j in range(1, 16):
                    group_min = jnp.minimum(group_min, iv[j])
                for j in range(14, -1, -1):
                    first = jnp.where(iv[j] == 0, j, first)

                @pl.when(group_min == 0)
                def _():
                    gbase = off + i * 16
                    pos = gbase + lax.iota(jnp.int32, 16)
                    targets = jnp.where(is_pad, pos, gbase + first)
                    pltpu.async_copy(zeros_v, out_hbm.at[targets], sem).wait()

                return 0

            lax.fori_loop(0, CHUNK // 16, fix_group, 0)

        return 0

    lax.fori_loop(0, NCHUNK, chunk_body, 0)


def kernel(inputs, table):
    idx = inputs.reshape(-1).astype(jnp.int32)
    out = _embed(idx, table)
    return out.reshape(BATCH, SEQ, DIM)


# speedup vs baseline: 9.2908x; 1.8036x over previous
import functools

import jax
import jax.numpy as jnp
from jax import lax
from jax.experimental import pallas as pl
from jax.experimental.pallas import tpu as pltpu
from jax.experimental.pallas import tpu_sc as plsc

VOCAB = 100000
DIM = 128
BATCH = 4096
SEQ = 200

NC = 2
NS = 16
NW = NC * NS

B = BATCH * SEQ
B_PER_W = B // NW
CHUNK = 128
NCHUNK = B_PER_W // CHUNK
NBUF = 4
NOUTER = NCHUNK // NBUF

_mesh = plsc.VectorSubcoreMesh(core_axis_name="c", subcore_axis_name="s")


@functools.partial(
    pl.kernel,
    mesh=_mesh,
    out_type=jax.ShapeDtypeStruct((B, DIM), jnp.float32),
    scratch_types=[
        pltpu.VMEM((NCHUNK, CHUNK), jnp.int32),
        pltpu.VMEM((NBUF, CHUNK, DIM), jnp.float32),
        pltpu.VMEM((16, DIM), jnp.float32),
        pltpu.SemaphoreType.DMA,
        pltpu.SemaphoreType.DMA,
        pltpu.SemaphoreType.DMA,
        pltpu.SemaphoreType.DMA,
        pltpu.SemaphoreType.DMA,
        pltpu.SemaphoreType.DMA,
        pltpu.SemaphoreType.DMA,
        pltpu.SemaphoreType.DMA,
        pltpu.SemaphoreType.DMA,
    ],
)
def _embed(idx_hbm, table_hbm, out_hbm, idx_v, bufs, zeros_v,
           g0, g1, g2, g3, s0, s1, s2, s3, zsem):
    gsem = (g0, g1, g2, g3)
    ssem = (s0, s1, s2, s3)
    wid = lax.axis_index("s") * NC + lax.axis_index("c")
    crow = wid * NCHUNK
    base = wid * B_PER_W

    pltpu.sync_copy(idx_hbm.at[pl.ds(crow, NCHUNK)], idx_v)

    zvec = jnp.zeros((16,), jnp.float32)

    def zinit(r, _):
        for c in range(DIM // 16):
            zeros_v[r, pl.ds(c * 16, 16)] = zvec
        return 0

    lax.fori_loop(0, 16, zinit, 0)

    def fire_gather(g, b):
        pltpu.async_copy(table_hbm.at[idx_v.at[g]], bufs.at[b], gsem[b])

    def pad_fix(g):
        def min_groups(i, acc):
            iv = idx_v[g, pl.ds(i * 16, 16)]
            return jnp.minimum(acc, iv)

        idx_min = lax.fori_loop(
            0, CHUNK // 16, min_groups, jnp.full((16,), VOCAB, jnp.int32)
        )
        chunk_min = idx_min[0]
        for j in range(1, 16):
            chunk_min = jnp.minimum(chunk_min, idx_min[j])

        @pl.when(chunk_min == 0)
        def _():
            def fix_group(i, _):
                iv = idx_v[g, pl.ds(i * 16, 16)]
                is_pad = iv == 0
                group_min = iv[0]
                first = jnp.where(iv[15] == 0, 15, 16)
                for j in range(1, 16):
                    group_min = jnp.minimum(group_min, iv[j])
                for j in range(14, -1, -1):
                    first = jnp.where(iv[j] == 0, j, first)

                @pl.when(group_min == 0)
                def _():
                    gbase = base + g * CHUNK + i * 16
                    pos = gbase + lax.iota(jnp.int32, 16)
                    targets = jnp.where(is_pad, pos, gbase + first)
                    pltpu.async_copy(
                        zeros_v, out_hbm.at[targets], zsem
                    ).wait()

                return 0

            lax.fori_loop(0, CHUNK // 16, fix_group, 0)

    for b in range(NBUF):
        fire_gather(b, b)

    def outer(t, _):
        for b in range(NBUF):
            g = t * NBUF + b
            pltpu.make_async_copy(
                table_hbm.at[idx_v.at[g]], bufs.at[b], gsem[b]
            ).wait()
            pltpu.async_copy(
                bufs.at[b], out_hbm.at[pl.ds(base + g * CHUNK, CHUNK)],
                ssem[b],
            )
        for b in range(NBUF):
            g = t * NBUF + b
            pltpu.make_async_copy(
                bufs.at[b], out_hbm.at[pl.ds(base + g * CHUNK, CHUNK)],
                ssem[b],
            ).wait()
            pad_fix(g)
            gn = g + NBUF

            @pl.when(gn < NCHUNK)
            def _():
                fire_gather(gn, b)

        return 0

    lax.fori_loop(0, NOUTER, outer, 0)


def kernel(inputs, table):
    idx = inputs.reshape(B // CHUNK, CHUNK).astype(jnp.int32)
    out = _embed(idx, table)
    return out.reshape(BATCH, SEQ, DIM)
